# Initial kernel scaffold; baseline (speedup 1.0000x reference)
import jax
import jax.numpy as jnp
from jax.experimental import pallas as pl


def kernel(pred, target):
    raise NotImplementedError("write your pallas kernel here")


# speedup vs baseline: 2.6671x; 2.6671x over previous
import functools
import jax
import jax.numpy as jnp
from jax.experimental import pallas as pl
from jax.experimental.pallas import tpu as pltpu


def _compare_exchange_sublane(x, k, mj, rb):
    rows, lanes = x.shape
    j = 1 << mj
    rhi = rows // (2 * j)
    v = x.reshape(rhi, 2, j, lanes)
    a = v[:, 0]
    b = v[:, 1]
    mn = jnp.minimum(a, b)
    mx = jnp.maximum(a, b)
    if k < rb:
        q = jax.lax.broadcasted_iota(jnp.int32, (rhi, j, lanes), 0)
        asc = ((q >> (k - mj - 1)) & 1) == 1
    else:
        l = jax.lax.broadcasted_iota(jnp.int32, (rhi, j, lanes), 2)
        asc = ((l >> (k - rb)) & 1) == 1
    na = jnp.where(asc, mn, mx)
    nb = jnp.where(asc, mx, mn)
    return jnp.stack([na, nb], axis=1).reshape(rows, lanes)


def _compare_exchange_lane(x, k, md, rb):
    rows, lanes = x.shape
    d = 1 << md
    rolled_p = jnp.concatenate([x[:, lanes - d:], x[:, :lanes - d]], axis=1)
    rolled_m = jnp.concatenate([x[:, d:], x[:, :d]], axis=1)
    l = jax.lax.broadcasted_iota(jnp.int32, (rows, lanes), 1)
    pbit = ((l >> md) & 1) == 1
    p = jnp.where(pbit, rolled_p, rolled_m)
    asc = ((l >> (k - rb)) & 1) == 1
    want_min = jnp.logical_xor(asc, pbit)
    return jnp.where(want_min, jnp.minimum(x, p), jnp.maximum(x, p))


def _bitonic_sort_desc(x, logn, rb):
    for k in range(1, logn + 1):
        for mj in range(k - 1, -1, -1):
            if mj < rb:
                x = _compare_exchange_sublane(x, k, mj, rb)
            else:
                x = _compare_exchange_lane(x, k, mj - rb, rb)
    return x


def _cumsum_rows(x):
    rows, lanes = x.shape
    s = 1
    while s < rows:
        shifted = jnp.concatenate(
            [jnp.zeros((s, lanes), x.dtype), x[: rows - s]], axis=0)
        x = x + shifted
        s *= 2
    return x


def _cumsum_lanes_excl(x):
    rows, lanes = x.shape
    x = jnp.concatenate([jnp.zeros((rows, 1), x.dtype), x[:, : lanes - 1]],
                        axis=1)
    s = 1
    while s < lanes:
        shifted = jnp.concatenate(
            [jnp.zeros((rows, s), x.dtype), x[:, : lanes - s]], axis=1)
        x = x + shifted
        s *= 2
    return x


def _lovasz_kernel(pred_ref, tgt_ref, out_ref, *, logn, rb, num_classes):
    c = pl.program_id(0)
    p = pred_ref[0]
    t = tgt_ref[...]
    rows, lanes = p.shape

    fg = (t == c)
    e = jnp.where(fg, 1.0 - p, p)
    bits = jax.lax.bitcast_convert_type(e, jnp.int32)
    key = (bits << 1) | fg.astype(jnp.int32)

    key = _bitonic_sort_desc(key, logn, rb)

    fs = (key & 1).astype(jnp.float32)
    es = jax.lax.bitcast_convert_type(
        jax.lax.shift_right_logical(key, 1), jnp.float32)

    g = jnp.sum(fs)
    csum = _cumsum_rows(fs)
    col_tot = csum[rows - 1:rows, :]
    col_pref = _cumsum_lanes_excl(col_tot)
    k_arr = csum + col_pref

    row_i = jax.lax.broadcasted_iota(jnp.int32, (rows, lanes), 0)
    lane_i = jax.lax.broadcasted_iota(jnp.int32, (rows, lanes), 1)
    n_arr = (lane_i * rows + row_i + 1).astype(jnp.float32)

    J = n_arr / (g + n_arr - k_arr)

    top_next = jnp.concatenate(
        [es[0:1, 1:], jnp.zeros((1, 1), jnp.float32)], axis=1)
    e_next = jnp.concatenate([es[1:], top_next], axis=0)

    loss_c = jnp.sum(J * (es - e_next))

    @pl.when(c == 0)
    def _():
        out_ref[...] = jnp.zeros_like(out_ref)

    out_ref[...] += loss_c * (1.0 / num_classes)


@jax.jit
def kernel(pred, target):
    P, C = pred.shape
    lanes = 128
    rows = P // lanes
    logn = P.bit_length() - 1
    rb = rows.bit_length() - 1

    pred_t = pred.T.reshape(C, rows, lanes)
    tgt = target.astype(jnp.int32).reshape(rows, lanes)

    out = pl.pallas_call(
        functools.partial(_lovasz_kernel, logn=logn, rb=rb, num_classes=C),
        grid=(C,),
        in_specs=[
            pl.BlockSpec((1, rows, lanes), lambda c: (c, 0, 0)),
            pl.BlockSpec((rows, lanes), lambda c: (0, 0)),
        ],
        out_specs=pl.BlockSpec((8, 128), lambda c: (0, 0)),
        out_shape=jax.ShapeDtypeStruct((8, 128), jnp.float32),
    )(pred_t, tgt)
    return out[0, 0]


# speedup vs baseline: 4.1079x; 1.5402x over previous
import functools
import jax
import jax.numpy as jnp
from jax.experimental import pallas as pl
from jax.experimental.pallas import tpu as pltpu


def _roll(x, dist, axis):
    n = x.shape[axis]
    d = dist % n
    if axis == 0:
        return jnp.concatenate([x[n - d:, :], x[: n - d, :]], axis=0)
    return jnp.concatenate([x[:, n - d:], x[:, : n - d]], axis=1)


def _lovasz_kernel(pred_ref, tgt_ref, out_ref, *, logn, rb, lanes):
    c = pl.program_id(0)
    p = pred_ref[0]
    t = tgt_ref[...]
    rows = p.shape[0]

    fg = (t == c)
    e = jnp.where(fg, 1.0 - p, p)
    bits = jax.lax.bitcast_convert_type(e, jnp.int32)
    x = (bits << 1) | fg.astype(jnp.int32)

    row_i = jax.lax.broadcasted_iota(jnp.int32, (rows, lanes), 0)
    lane_i = jax.lax.broadcasted_iota(jnp.int32, (rows, lanes), 1)

    def idx_bit(b):
        if b < rb:
            return (row_i >> b) & 1
        return (lane_i >> (b - rb)) & 1

    pbit = [idx_bit(m) == 1 for m in range(logn)]
    flip = [idx_bit(k) - 1 for k in range(1, logn + 1)]

    x = x ^ flip[0]
    for k in range(1, logn + 1):
        if k > 1:
            x = x ^ (flip[k - 2] ^ flip[k - 1])
        for m in range(k - 1, -1, -1):
            if m < rb:
                axis, dist = 0, 1 << m
            else:
                axis, dist = 1, 1 << (m - rb)
            partner = jnp.where(pbit[m], _roll(x, dist, axis),
                                _roll(x, -dist, axis))
            x = jnp.where(pbit[m], jnp.maximum(x, partner),
                          jnp.minimum(x, partner))
    x = x ^ flip[logn - 1]

    fs = (x & 1).astype(jnp.float32)
    es = jax.lax.bitcast_convert_type(
        jax.lax.shift_right_logical(x, 1), jnp.float32)

    g = jnp.sum(fs)
    csum = fs
    s = 1
    while s < rows:
        csum = csum + jnp.concatenate(
            [jnp.zeros((s, lanes), jnp.float32), csum[: rows - s]], axis=0)
        s *= 2
    col_tot = csum[rows - 1:rows, :]
    cp = jnp.concatenate([jnp.zeros((1, 1), jnp.float32),
                          col_tot[:, : lanes - 1]], axis=1)
    s = 1
    while s < lanes:
        cp = cp + jnp.concatenate(
            [jnp.zeros((1, s), jnp.float32), cp[:, : lanes - s]], axis=1)
        s *= 2
    k_arr = csum + cp

    n_arr = (lane_i * rows + row_i + 1).astype(jnp.float32)
    J = n_arr / (g + n_arr - k_arr)

    top_next = jnp.concatenate(
        [es[0:1, 1:], jnp.zeros((1, 1), jnp.float32)], axis=1)
    e_next = jnp.concatenate([es[1:], top_next], axis=0)

    loss_c = jnp.sum(J * (es - e_next))
    out_ref[...] = jnp.full(out_ref.shape, loss_c, jnp.float32)


@jax.jit
def kernel(pred, target):
    P, C = pred.shape
    lanes = 128
    rows = P // lanes
    logn = P.bit_length() - 1
    rb = rows.bit_length() - 1

    pred_t = pred.T.reshape(C, rows, lanes)
    tgt = target.astype(jnp.int32).reshape(rows, lanes)

    out = pl.pallas_call(
        functools.partial(_lovasz_kernel, logn=logn, rb=rb, lanes=lanes),
        grid=(C,),
        in_specs=[
            pl.BlockSpec((1, rows, lanes), lambda c: (c, 0, 0)),
            pl.BlockSpec((rows, lanes), lambda c: (0, 0)),
        ],
        out_specs=pl.BlockSpec((1, 8, 128), lambda c: (c, 0, 0)),
        out_shape=jax.ShapeDtypeStruct((C, 8, 128), jnp.float32),
        compiler_params=pltpu.CompilerParams(
            dimension_semantics=("parallel",)),
    )(pred_t, tgt)
    return jnp.mean(out[:, 0, 0])


# speedup vs baseline: 4.7541x; 1.1573x over previous
import functools
import jax
import jax.numpy as jnp
from jax.experimental import pallas as pl
from jax.experimental.pallas import tpu as pltpu


def _roll(x, dist, axis):
    n = x.shape[axis]
    d = dist % n
    if axis == 0:
        return jnp.concatenate([x[n - d:, :], x[: n - d, :]], axis=0)
    return jnp.concatenate([x[:, n - d:], x[:, : n - d]], axis=1)


def _lovasz_kernel(pred_ref, tgt_ref, out_ref, *, logn, rb, lanes):
    c = pl.program_id(0)
    p = pred_ref[0]
    t = tgt_ref[...]
    rows = p.shape[0]

    fg = (t == c)
    e = jnp.where(fg, 1.0 - p, p)
    bits = jax.lax.bitcast_convert_type(e, jnp.int32)
    x = (bits << 1) | fg.astype(jnp.int32)

    row_i = jax.lax.broadcasted_iota(jnp.int32, (rows, lanes), 0)
    lane_i = jax.lax.broadcasted_iota(jnp.int32, (rows, lanes), 1)

    def idx_bit(b):
        if b < rb:
            return (row_i >> b) & 1
        return (lane_i >> (b - rb)) & 1

    pbit = [idx_bit(m) == 1 for m in range(logn)]
    flip = [idx_bit(k) - 1 for k in range(1, logn + 1)]

    x = x ^ flip[0]
    for k in range(1, logn + 1):
        if k > 1:
            x = x ^ (flip[k - 2] ^ flip[k - 1])
        for m in range(k - 1, -1, -1):
            if m < rb:
                axis, dist = 0, 1 << m
            else:
                axis, dist = 1, 1 << (m - rb)
            y = _roll(x, -dist, axis)
            mn = jnp.minimum(x, y)
            mx = jnp.maximum(x, y)
            x = jnp.where(pbit[m], _roll(mx, dist, axis), mn)
    x = x ^ flip[logn - 1]

    fs = (x & 1).astype(jnp.float32)
    es = jax.lax.bitcast_convert_type(
        jax.lax.shift_right_logical(x, 1), jnp.float32)

    g = jnp.sum(fs)
    csum = fs
    s = 1
    while s < rows:
        csum = csum + jnp.concatenate(
            [jnp.zeros((s, lanes), jnp.float32), csum[: rows - s]], axis=0)
        s *= 2
    col_tot = csum[rows - 1:rows, :]
    cp = jnp.concatenate([jnp.zeros((1, 1), jnp.float32),
                          col_tot[:, : lanes - 1]], axis=1)
    s = 1
    while s < lanes:
        cp = cp + jnp.concatenate(
            [jnp.zeros((1, s), jnp.float32), cp[:, : lanes - s]], axis=1)
        s *= 2
    k_arr = csum + cp

    n_arr = (lane_i * rows + row_i + 1).astype(jnp.float32)
    J = n_arr / (g + n_arr - k_arr)

    top_next = jnp.concatenate(
        [es[0:1, 1:], jnp.zeros((1, 1), jnp.float32)], axis=1)
    e_next = jnp.concatenate([es[1:], top_next], axis=0)

    loss_c = jnp.sum(J * (es - e_next))
    out_ref[...] = jnp.full(out_ref.shape, loss_c, jnp.float32)


@jax.jit
def kernel(pred, target):
    P, C = pred.shape
    lanes = 128
    rows = P // lanes
    logn = P.bit_length() - 1
    rb = rows.bit_length() - 1

    pred_t = pred.T.reshape(C, rows, lanes)
    tgt = target.astype(jnp.int32).reshape(rows, lanes)

    out = pl.pallas_call(
        functools.partial(_lovasz_kernel, logn=logn, rb=rb, lanes=lanes),
        grid=(C,),
        in_specs=[
            pl.BlockSpec((1, rows, lanes), lambda c: (c, 0, 0)),
            pl.BlockSpec((rows, lanes), lambda c: (0, 0)),
        ],
        out_specs=pl.BlockSpec((1, 8, 128), lambda c: (c, 0, 0)),
        out_shape=jax.ShapeDtypeStruct((C, 8, 128), jnp.float32),
        compiler_params=pltpu.CompilerParams(
            dimension_semantics=("parallel",)),
    )(pred_t, tgt)
    return jnp.mean(out[:, 0, 0])
